# baseline (speedup 1.0000x reference)
import functools

import jax
import jax.numpy as jnp
from jax import lax
from jax.experimental import pallas as pl
from jax.experimental.pallas import tpu as pltpu
from jax.experimental.pallas import tpu_sc as plsc

N = 10000
E = 320000
D_IN = 128
D_HID = 64
D_OUT = 128

NC = 2
NS = 16
NW = NC * NS
CK = 128
CH = 81
E_PAD = NW * CH * CK
N_ACC = 10112
ZROWS = N_ACC // NS
DEG_W = 16
DA = 64
KB = 1
NBURST = CH // KB
TROWS = N // NS
DEG_G = 8
EPW = E // NW
TAIL = CH * CK - EPW

_mesh = plsc.VectorSubcoreMesh(core_axis_name="c", subcore_axis_name="s")
_sc_params = pltpu.CompilerParams(use_tc_tiling_on_sc=False)


def _load_idx(ei_hbm, row, idx_v, w, junk):
    pltpu.sync_copy(ei_hbm.at[row, pl.ds(w * EPW, EPW)],
                    idx_v.at[pl.ds(0, EPW)])
    fill = jnp.full((16,), junk, jnp.int32)
    for k in range(TAIL // 16):
        idx_v[pl.ds(EPW + k * 16, 16)] = fill


def _deg_sc(edge_index, ones_hbm, zeros_hbm):
    @functools.partial(
        pl.kernel,
        out_type=jax.ShapeDtypeStruct((NC, N_ACC, DEG_W), jnp.float32),
        mesh=_mesh,
        scratch_types=[
            pltpu.VMEM((CH * CK,), jnp.int32),
            pltpu.VMEM((CK, DEG_W), jnp.float32),
            pltpu.VMEM_SHARED((N_ACC, DEG_W), jnp.float32),
            pltpu.SemaphoreType.DMA,
        ],
        compiler_params=_sc_params,
    )
    def deg_kernel(ei_hbm, ones_h, zeros_h, out_hbm, dst_v, ones_v, acc,
                   sem):
        c = lax.axis_index("c")
        s = lax.axis_index("s")
        w = c * NS + s
        pltpu.sync_copy(zeros_h.at[pl.ds(s * ZROWS, ZROWS)],
                        acc.at[pl.ds(s * ZROWS, ZROWS)])
        pltpu.sync_copy(ones_h, ones_v)
        _load_idx(ei_hbm, 1, dst_v, w, N)
        plsc.subcore_barrier()

        @pl.loop(0, CH - (CH % DEG_G), step=DEG_G)
        def _(j):
            for t in range(DEG_G):
                pltpu.async_copy(
                    ones_v, acc.at[dst_v.at[pl.ds((j + t) * CK, CK)]], sem,
                    add=True)
            for t in range(DEG_G):
                pltpu.make_async_copy(
                    ones_v, acc.at[dst_v.at[pl.ds((j + t) * CK, CK)]],
                    sem).wait()

        for j in range(CH - (CH % DEG_G), CH):
            pltpu.sync_copy(ones_v, acc.at[dst_v.at[pl.ds(j * CK, CK)]],
                            add=True)

        plsc.subcore_barrier()
        pltpu.sync_copy(acc.at[pl.ds(s * ZROWS, ZROWS)],
                        out_hbm.at[c, pl.ds(s * ZROWS, ZROWS)])

    return deg_kernel(edge_index, ones_hbm, zeros_hbm)


def _agg_sc(tables, edge_index, zeros_hbm):
    nh = tables.shape[0]

    @functools.partial(
        pl.kernel,
        out_type=jax.ShapeDtypeStruct((nh, NC, N_ACC, DA), jnp.float32),
        mesh=_mesh,
        scratch_types=[
            pltpu.VMEM((CH * CK,), jnp.int32),
            pltpu.VMEM((CH * CK,), jnp.int32),
            pltpu.VMEM((CK, DA), jnp.float32),
            pltpu.VMEM((CK, DA), jnp.float32),
            pltpu.VMEM((CK, DA), jnp.float32),
            pltpu.VMEM_SHARED((N_ACC, DA), jnp.float32),
            pltpu.VMEM_SHARED((N, DA), jnp.float32),
            pltpu.SemaphoreType.DMA,
            pltpu.SemaphoreType.DMA,
            pltpu.SemaphoreType.DMA,
            pltpu.SemaphoreType.DMA,
            pltpu.SemaphoreType.DMA,
            pltpu.SemaphoreType.DMA,
        ],
        compiler_params=_sc_params,
    )
    def agg_kernel(tables_hbm, ei_hbm, zeros_h, out_hbm,
                   src_v, dst_v, b0, b1, b2, acc, tab_s,
                   g0, g1, g2, s0, s1, s2):
        c = lax.axis_index("c")
        s = lax.axis_index("s")
        w = c * NS + s
        _load_idx(ei_hbm, 0, src_v, w, 0)
        _load_idx(ei_hbm, 1, dst_v, w, N)
        bufs = (b0, b1, b2)
        gsems = (g0, g1, g2)
        ssems = (s0, s1, s2)

        for h in range(nh):
            def fire_g(j, k):
                pltpu.async_copy(tab_s.at[src_v.at[pl.ds(j * CK, CK)]],
                                 bufs[k], gsems[k])

            def drain_g(k):
                pltpu.make_async_copy(tab_s.at[src_v.at[pl.ds(0, CK)]],
                                      bufs[k], gsems[k]).wait()

            def fire_s(j, k):
                pltpu.async_copy(bufs[k],
                                 acc.at[dst_v.at[pl.ds(j * CK, CK)]],
                                 ssems[k], add=True)

            def drain_s(k):
                pltpu.make_async_copy(bufs[k],
                                      acc.at[dst_v.at[pl.ds(0, CK)]],
                                      ssems[k]).wait()

            pltpu.sync_copy(zeros_h.at[pl.ds(s * ZROWS, ZROWS)],
                            acc.at[pl.ds(s * ZROWS, ZROWS)])
            pltpu.sync_copy(tables_hbm.at[h, pl.ds(s * TROWS, TROWS)],
                            tab_s.at[pl.ds(s * TROWS, TROWS)])
            plsc.subcore_barrier()

            fire_g(0, 0)
            fire_g(1, 1)

            @pl.loop(0, NBURST, step=3)
            def _(i):
                for k in range(3):
                    j = i + k
                    k2 = (k + 2) % 3
                    drain_g(k)
                    fire_s(j, k)

                    @pl.when(j == 0)
                    def _():
                        fire_g(2, 2)

                    @pl.when((j >= 1) & (j + 2 < NBURST))
                    def _():
                        drain_s(k2)
                        fire_g(j + 2, k2)

            for k in range(3):
                drain_s(k)

            plsc.subcore_barrier()
            pltpu.sync_copy(acc.at[pl.ds(s * ZROWS, ZROWS)],
                            out_hbm.at[h, c, pl.ds(s * ZROWS, ZROWS)])

    out = agg_kernel(tables, edge_index, zeros_hbm)
    return out[:, 0] + out[:, 1]


BN = 2000
GRID = N // BN


def _dinv_from(deg_ref):
    return lax.rsqrt(deg_ref[...] + 1.0)


def _tc_mm1_body(x_ref, w_ref, h_ref):
    h_ref[...] = jnp.dot(x_ref[...], w_ref[...],
                         preferred_element_type=jnp.float32)


def _tc_scale_body(h_ref, degp_ref, hs_ref):
    hs_ref[...] = h_ref[...] * _dinv_from(degp_ref)


def _tc2_body(hs1_ref, agg_ref, deg_ref, w2_ref, b1_ref, hs2_ref):
    dinv = _dinv_from(deg_ref)
    pre = (agg_ref[0] + hs1_ref[...]) * dinv + b1_ref[...]
    out1 = jnp.maximum(pre, 0.0)
    h2 = jnp.dot(out1, w2_ref[...], preferred_element_type=jnp.float32)
    hs2 = h2 * dinv
    hs2_ref[0] = hs2[:, :DA]
    hs2_ref[1] = hs2[:, DA:]


def _tc3_body(hs2_ref, agg_ref, deg_ref, b2_ref, out_ref):
    dinv = _dinv_from(deg_ref)
    lo = (agg_ref[0] + hs2_ref[0]) * dinv
    hi = (agg_ref[1] + hs2_ref[1]) * dinv
    out_ref[...] = jnp.concatenate([lo, hi], axis=1) + b2_ref[...]


_deg_spec = pl.BlockSpec((BN, 1), lambda i: (i, 0))


def kernel(x, edge_index, W1, b1, W2, b2):
    ones16 = jnp.ones((CK, DEG_W), jnp.float32)
    zeros_deg = jnp.zeros((N_ACC, DEG_W), jnp.float32)
    zeros_da = jnp.zeros((N_ACC, DA), jnp.float32)

    degp = _deg_sc(edge_index, ones16, zeros_deg)
    deg = degp[0, :, :1] + degp[1, :, :1]

    h1 = pl.pallas_call(
        _tc_mm1_body,
        out_shape=jax.ShapeDtypeStruct((N, D_HID), jnp.float32),
    )(x, W1)

    hs1 = pl.pallas_call(
        _tc_scale_body,
        grid=(GRID,),
        in_specs=[
            pl.BlockSpec((BN, D_HID), lambda i: (i, 0)),
            _deg_spec,
        ],
        out_specs=pl.BlockSpec((BN, D_HID), lambda i: (i, 0)),
        out_shape=jax.ShapeDtypeStruct((N, D_HID), jnp.float32),
    )(h1, deg)

    agg1 = _agg_sc(hs1.reshape(1, N, DA), edge_index, zeros_da)

    hs2 = pl.pallas_call(
        _tc2_body,
        grid=(GRID,),
        in_specs=[
            pl.BlockSpec((BN, D_HID), lambda i: (i, 0)),
            pl.BlockSpec((1, BN, DA), lambda i: (0, i, 0)),
            _deg_spec,
            pl.BlockSpec((D_HID, D_OUT), lambda i: (0, 0)),
            pl.BlockSpec((1, D_HID), lambda i: (0, 0)),
        ],
        out_specs=pl.BlockSpec((2, BN, DA), lambda i: (0, i, 0)),
        out_shape=jax.ShapeDtypeStruct((2, N, DA), jnp.float32),
    )(hs1, agg1, deg, W2, b1.reshape(1, D_HID))

    agg2 = _agg_sc(hs2, edge_index, zeros_da)

    out = pl.pallas_call(
        _tc3_body,
        grid=(GRID,),
        in_specs=[
            pl.BlockSpec((2, BN, DA), lambda i: (0, i, 0)),
            pl.BlockSpec((2, BN, DA), lambda i: (0, i, 0)),
            _deg_spec,
            pl.BlockSpec((1, D_OUT), lambda i: (0, 0)),
        ],
        out_specs=pl.BlockSpec((BN, D_OUT), lambda i: (i, 0)),
        out_shape=jax.ShapeDtypeStruct((N, D_OUT), jnp.float32),
    )(hs2, agg2, deg, b2.reshape(1, D_OUT))

    return out


# speedup vs baseline: 1.0589x; 1.0589x over previous
import functools

import jax
import jax.numpy as jnp
from jax import lax
from jax.experimental import pallas as pl
from jax.experimental.pallas import tpu as pltpu
from jax.experimental.pallas import tpu_sc as plsc

N = 10000
E = 320000
D_IN = 128
D_HID = 64
D_OUT = 128

NC = 2
NS = 16
NW = NC * NS
CK = 128
CH = 81
E_PAD = NW * CH * CK
N_ACC = 10112
ZROWS = N_ACC // NS
DEG_W = 8
DA = 64
KB = 1
NBURST = CH // KB
TROWS = N // NS
DEG_G = 8
EPW = E // NW
TAIL = CH * CK - EPW

_mesh = plsc.VectorSubcoreMesh(core_axis_name="c", subcore_axis_name="s")
_sc_params = pltpu.CompilerParams(use_tc_tiling_on_sc=False)


def _load_idx(ei_hbm, row, idx_v, w, junk):
    pltpu.sync_copy(ei_hbm.at[row, pl.ds(w * EPW, EPW)],
                    idx_v.at[pl.ds(0, EPW)])
    fill = jnp.full((16,), junk, jnp.int32)
    for k in range(TAIL // 16):
        idx_v[pl.ds(EPW + k * 16, 16)] = fill


def _deg_sc(edge_index, ones_hbm, zeros_hbm):
    @functools.partial(
        pl.kernel,
        out_type=jax.ShapeDtypeStruct((NC, N_ACC, DEG_W), jnp.float32),
        mesh=_mesh,
        scratch_types=[
            pltpu.VMEM((CH * CK,), jnp.int32),
            pltpu.VMEM((CK, DEG_W), jnp.float32),
            pltpu.VMEM_SHARED((N_ACC, DEG_W), jnp.float32),
            pltpu.SemaphoreType.DMA,
        ],
        compiler_params=_sc_params,
    )
    def deg_kernel(ei_hbm, ones_h, zeros_h, out_hbm, dst_v, ones_v, acc,
                   sem):
        c = lax.axis_index("c")
        s = lax.axis_index("s")
        w = c * NS + s
        pltpu.sync_copy(zeros_h.at[pl.ds(s * ZROWS, ZROWS)],
                        acc.at[pl.ds(s * ZROWS, ZROWS)])
        pltpu.sync_copy(ones_h, ones_v)
        _load_idx(ei_hbm, 1, dst_v, w, N)
        plsc.subcore_barrier()

        @pl.loop(0, CH - (CH % DEG_G), step=DEG_G)
        def _(j):
            for t in range(DEG_G):
                pltpu.async_copy(
                    ones_v, acc.at[dst_v.at[pl.ds((j + t) * CK, CK)]], sem,
                    add=True)
            for t in range(DEG_G):
                pltpu.make_async_copy(
                    ones_v, acc.at[dst_v.at[pl.ds((j + t) * CK, CK)]],
                    sem).wait()

        for j in range(CH - (CH % DEG_G), CH):
            pltpu.sync_copy(ones_v, acc.at[dst_v.at[pl.ds(j * CK, CK)]],
                            add=True)

        plsc.subcore_barrier()
        pltpu.sync_copy(acc.at[pl.ds(s * ZROWS, ZROWS)],
                        out_hbm.at[c, pl.ds(s * ZROWS, ZROWS)])

    return deg_kernel(edge_index, ones_hbm, zeros_hbm)


def _agg_sc(tables, edge_index, zeros_hbm):
    nh = tables.shape[0]

    @functools.partial(
        pl.kernel,
        out_type=jax.ShapeDtypeStruct((nh, NC, N_ACC, DA), jnp.float32),
        mesh=_mesh,
        scratch_types=[
            pltpu.VMEM((CH * CK,), jnp.int32),
            pltpu.VMEM((CH * CK,), jnp.int32),
            pltpu.VMEM((CK, DA), jnp.float32),
            pltpu.VMEM((CK, DA), jnp.float32),
            pltpu.VMEM((CK, DA), jnp.float32),
            pltpu.VMEM_SHARED((N_ACC, DA), jnp.float32),
            pltpu.VMEM_SHARED((N, DA), jnp.float32),
            pltpu.SemaphoreType.DMA,
            pltpu.SemaphoreType.DMA,
            pltpu.SemaphoreType.DMA,
            pltpu.SemaphoreType.DMA,
            pltpu.SemaphoreType.DMA,
            pltpu.SemaphoreType.DMA,
        ],
        compiler_params=_sc_params,
    )
    def agg_kernel(tables_hbm, ei_hbm, zeros_h, out_hbm,
                   src_v, dst_v, b0, b1, b2, acc, tab_s,
                   g0, g1, g2, s0, s1, s2):
        c = lax.axis_index("c")
        s = lax.axis_index("s")
        w = c * NS + s
        _load_idx(ei_hbm, 0, src_v, w, 0)
        _load_idx(ei_hbm, 1, dst_v, w, N)
        bufs = (b0, b1, b2)
        gsems = (g0, g1, g2)
        ssems = (s0, s1, s2)

        for h in range(nh):
            def fire_g(j, k):
                pltpu.async_copy(tab_s.at[src_v.at[pl.ds(j * CK, CK)]],
                                 bufs[k], gsems[k])

            def drain_g(k):
                pltpu.make_async_copy(tab_s.at[src_v.at[pl.ds(0, CK)]],
                                      bufs[k], gsems[k]).wait()

            def fire_s(j, k):
                pltpu.async_copy(bufs[k],
                                 acc.at[dst_v.at[pl.ds(j * CK, CK)]],
                                 ssems[k], add=True)

            def drain_s(k):
                pltpu.make_async_copy(bufs[k],
                                      acc.at[dst_v.at[pl.ds(0, CK)]],
                                      ssems[k]).wait()

            pltpu.sync_copy(zeros_h.at[pl.ds(s * ZROWS, ZROWS)],
                            acc.at[pl.ds(s * ZROWS, ZROWS)])
            pltpu.sync_copy(tables_hbm.at[h, pl.ds(s * TROWS, TROWS)],
                            tab_s.at[pl.ds(s * TROWS, TROWS)])
            plsc.subcore_barrier()

            fire_g(0, 0)
            fire_g(1, 1)

            @pl.loop(0, NBURST, step=3)
            def _(i):
                for k in range(3):
                    j = i + k
                    k2 = (k + 2) % 3
                    drain_g(k)
                    fire_s(j, k)

                    @pl.when(j == 0)
                    def _():
                        fire_g(2, 2)

                    @pl.when((j >= 1) & (j + 2 < NBURST))
                    def _():
                        drain_s(k2)
                        fire_g(j + 2, k2)

            for k in range(3):
                drain_s(k)

            plsc.subcore_barrier()
            pltpu.sync_copy(acc.at[pl.ds(s * ZROWS, ZROWS)],
                            out_hbm.at[h, c, pl.ds(s * ZROWS, ZROWS)])

    return agg_kernel(tables, edge_index, zeros_hbm)


BN = 2000
GRID = N // BN


def _dinv_from(degp_ref):
    deg = degp_ref[0, :, :1] + degp_ref[1, :, :1] + 1.0
    return lax.rsqrt(deg)


def _tc_mm1_body(x_ref, w_ref, h_ref):
    h_ref[...] = jnp.dot(x_ref[...], w_ref[...],
                         preferred_element_type=jnp.float32)


def _tc_scale_body(h_ref, degp_ref, hs_ref):
    hs_ref[...] = h_ref[...] * _dinv_from(degp_ref)


def _tc2_body(hs1_ref, aggp_ref, degp_ref, w2_ref, b1_ref, hs2_ref):
    dinv = _dinv_from(degp_ref)
    pre = (aggp_ref[0, 0] + aggp_ref[0, 1] + hs1_ref[...]) * dinv \
        + b1_ref[...]
    out1 = jnp.maximum(pre, 0.0)
    h2 = jnp.dot(out1, w2_ref[...], preferred_element_type=jnp.float32)
    hs2 = h2 * dinv
    hs2_ref[0] = hs2[:, :DA]
    hs2_ref[1] = hs2[:, DA:]


def _tc3_body(hs2_ref, aggp_ref, degp_ref, b2_ref, out_ref):
    dinv = _dinv_from(degp_ref)
    lo = (aggp_ref[0, 0] + aggp_ref[0, 1] + hs2_ref[0]) * dinv
    hi = (aggp_ref[1, 0] + aggp_ref[1, 1] + hs2_ref[1]) * dinv
    out_ref[...] = jnp.concatenate([lo, hi], axis=1) + b2_ref[...]


_deg_spec = pl.BlockSpec((NC, BN, DEG_W), lambda i: (0, i, 0))


def kernel(x, edge_index, W1, b1, W2, b2):
    ones16 = jnp.ones((CK, DEG_W), jnp.float32)
    zeros_deg = jnp.zeros((N_ACC, DEG_W), jnp.float32)
    zeros_da = jnp.zeros((N_ACC, DA), jnp.float32)

    degp = _deg_sc(edge_index, ones16, zeros_deg)

    h1 = pl.pallas_call(
        _tc_mm1_body,
        out_shape=jax.ShapeDtypeStruct((N, D_HID), jnp.float32),
    )(x, W1)

    hs1 = pl.pallas_call(
        _tc_scale_body,
        grid=(GRID,),
        in_specs=[
            pl.BlockSpec((BN, D_HID), lambda i: (i, 0)),
            _deg_spec,
        ],
        out_specs=pl.BlockSpec((BN, D_HID), lambda i: (i, 0)),
        out_shape=jax.ShapeDtypeStruct((N, D_HID), jnp.float32),
    )(h1, degp)

    agg1 = _agg_sc(hs1.reshape(1, N, DA), edge_index, zeros_da)

    hs2 = pl.pallas_call(
        _tc2_body,
        grid=(GRID,),
        in_specs=[
            pl.BlockSpec((BN, D_HID), lambda i: (i, 0)),
            pl.BlockSpec((1, NC, BN, DA), lambda i: (0, 0, i, 0)),
            _deg_spec,
            pl.BlockSpec((D_HID, D_OUT), lambda i: (0, 0)),
            pl.BlockSpec((1, D_HID), lambda i: (0, 0)),
        ],
        out_specs=pl.BlockSpec((2, BN, DA), lambda i: (0, i, 0)),
        out_shape=jax.ShapeDtypeStruct((2, N, DA), jnp.float32),
    )(hs1, agg1, degp, W2, b1.reshape(1, D_HID))

    agg2 = _agg_sc(hs2, edge_index, zeros_da)

    out = pl.pallas_call(
        _tc3_body,
        grid=(GRID,),
        in_specs=[
            pl.BlockSpec((2, BN, DA), lambda i: (0, i, 0)),
            pl.BlockSpec((2, NC, BN, DA), lambda i: (0, 0, i, 0)),
            _deg_spec,
            pl.BlockSpec((1, D_OUT), lambda i: (0, 0)),
        ],
        out_specs=pl.BlockSpec((BN, D_OUT), lambda i: (i, 0)),
        out_shape=jax.ShapeDtypeStruct((N, D_OUT), jnp.float32),
    )(hs2, agg2, degp, b2.reshape(1, D_OUT))

    return out


# speedup vs baseline: 1.1261x; 1.0635x over previous
import functools

import jax
import jax.numpy as jnp
from jax import lax
from jax.experimental import pallas as pl
from jax.experimental.pallas import tpu as pltpu
from jax.experimental.pallas import tpu_sc as plsc

N = 10000
E = 320000
D_IN = 128
D_HID = 64
D_OUT = 128

NC = 2
NS = 16
NW = NC * NS
CK = 128
CH = 79
E_PAD = NW * CH * CK
N_ACC = 10112
ZROWS = N_ACC // NS
DEG_W = 8
DA = 64
NBURST = CH
LOOPB = NBURST - (NBURST % 3)
TROWS = N // NS
DEG_G = 8
EPW = E // NW
TAIL = CH * CK - EPW

_mesh = plsc.VectorSubcoreMesh(core_axis_name="c", subcore_axis_name="s")
_sc_params = pltpu.CompilerParams(use_tc_tiling_on_sc=False)


def _load_idx(ei_hbm, row, idx_v, w, junk):
    pltpu.sync_copy(ei_hbm.at[row, pl.ds(w * EPW, EPW)],
                    idx_v.at[pl.ds(0, EPW)])
    fill = jnp.full((16,), junk, jnp.int32)
    for k in range(TAIL // 16):
        idx_v[pl.ds(EPW + k * 16, 16)] = fill


def _deg_sc(edge_index, ones_hbm, zeros_hbm):
    @functools.partial(
        pl.kernel,
        out_type=jax.ShapeDtypeStruct((NC, N_ACC, DEG_W), jnp.float32),
        mesh=_mesh,
        scratch_types=[
            pltpu.VMEM((CH * CK,), jnp.int32),
            pltpu.VMEM((CK, DEG_W), jnp.float32),
            pltpu.VMEM_SHARED((N_ACC, DEG_W), jnp.float32),
            pltpu.SemaphoreType.DMA,
        ],
        compiler_params=_sc_params,
    )
    def deg_kernel(ei_hbm, ones_h, zeros_h, out_hbm, dst_v, ones_v, acc,
                   sem):
        c = lax.axis_index("c")
        s = lax.axis_index("s")
        w = c * NS + s
        pltpu.sync_copy(zeros_h.at[pl.ds(s * ZROWS, ZROWS)],
                        acc.at[pl.ds(s * ZROWS, ZROWS)])
        pltpu.sync_copy(ones_h, ones_v)
        _load_idx(ei_hbm, 1, dst_v, w, N)
        plsc.subcore_barrier()

        @pl.loop(0, CH - (CH % DEG_G), step=DEG_G)
        def _(j):
            for t in range(DEG_G):
                pltpu.async_copy(
                    ones_v, acc.at[dst_v.at[pl.ds((j + t) * CK, CK)]], sem,
                    add=True)
            for t in range(DEG_G):
                pltpu.make_async_copy(
                    ones_v, acc.at[dst_v.at[pl.ds((j + t) * CK, CK)]],
                    sem).wait()

        for j in range(CH - (CH % DEG_G), CH):
            pltpu.sync_copy(ones_v, acc.at[dst_v.at[pl.ds(j * CK, CK)]],
                            add=True)

        plsc.subcore_barrier()
        pltpu.sync_copy(acc.at[pl.ds(s * ZROWS, ZROWS)],
                        out_hbm.at[c, pl.ds(s * ZROWS, ZROWS)])

    return deg_kernel(edge_index, ones_hbm, zeros_hbm)


def _agg_sc(tables, edge_index, zeros_hbm):
    nh = tables.shape[0]

    @functools.partial(
        pl.kernel,
        out_type=jax.ShapeDtypeStruct((nh, NC, N_ACC, DA), jnp.float32),
        mesh=_mesh,
        scratch_types=[
            pltpu.VMEM((CH * CK,), jnp.int32),
            pltpu.VMEM((CH * CK,), jnp.int32),
            pltpu.VMEM((CK, DA), jnp.float32),
            pltpu.VMEM((CK, DA), jnp.float32),
            pltpu.VMEM((CK, DA), jnp.float32),
            pltpu.VMEM_SHARED((N_ACC, DA), jnp.float32),
            pltpu.VMEM_SHARED((N, DA), jnp.float32),
            pltpu.SemaphoreType.DMA,
            pltpu.SemaphoreType.DMA,
            pltpu.SemaphoreType.DMA,
            pltpu.SemaphoreType.DMA,
            pltpu.SemaphoreType.DMA,
            pltpu.SemaphoreType.DMA,
        ],
        compiler_params=_sc_params,
    )
    def agg_kernel(tables_hbm, ei_hbm, zeros_h, out_hbm,
                   src_v, dst_v, b0, b1, b2, acc, tab_s,
                   g0, g1, g2, s0, s1, s2):
        c = lax.axis_index("c")
        s = lax.axis_index("s")
        w = c * NS + s
        _load_idx(ei_hbm, 0, src_v, w, 0)
        _load_idx(ei_hbm, 1, dst_v, w, N)
        bufs = (b0, b1, b2)
        gsems = (g0, g1, g2)
        ssems = (s0, s1, s2)

        for h in range(nh):
            def fire_g(j, k):
                pltpu.async_copy(tab_s.at[src_v.at[pl.ds(j * CK, CK)]],
                                 bufs[k], gsems[k])

            def drain_g(k):
                pltpu.make_async_copy(tab_s.at[src_v.at[pl.ds(0, CK)]],
                                      bufs[k], gsems[k]).wait()

            def fire_s(j, k):
                pltpu.async_copy(bufs[k],
                                 acc.at[dst_v.at[pl.ds(j * CK, CK)]],
                                 ssems[k], add=True)

            def drain_s(k):
                pltpu.make_async_copy(bufs[k],
                                      acc.at[dst_v.at[pl.ds(0, CK)]],
                                      ssems[k]).wait()

            pltpu.sync_copy(zeros_h.at[pl.ds(s * ZROWS, ZROWS)],
                            acc.at[pl.ds(s * ZROWS, ZROWS)])
            pltpu.sync_copy(tables_hbm.at[h, pl.ds(s * TROWS, TROWS)],
                            tab_s.at[pl.ds(s * TROWS, TROWS)])
            plsc.subcore_barrier()

            fire_g(0, 0)
            fire_g(1, 1)

            @pl.loop(0, LOOPB, step=3)
            def _(i):
                for k in range(3):
                    j = i + k
                    k2 = (k + 2) % 3
                    drain_g(k)
                    fire_s(j, k)

                    @pl.when(j == 0)
                    def _():
                        fire_g(2, 2)

                    @pl.when((j >= 1) & (j + 2 < NBURST))
                    def _():
                        drain_s(k2)
                        fire_g(j + 2, k2)

            for b in range(LOOPB, NBURST):
                drain_g(b % 3)
                fire_s(b, b % 3)
            for b in range(NBURST - 3, NBURST):
                drain_s(b % 3)

            plsc.subcore_barrier()
            pltpu.sync_copy(acc.at[pl.ds(s * ZROWS, ZROWS)],
                            out_hbm.at[h, c, pl.ds(s * ZROWS, ZROWS)])

    return agg_kernel(tables, edge_index, zeros_hbm)


BN = 2000
GRID = N // BN


def _dinv_from(degp_ref):
    deg = degp_ref[0, :, :1] + degp_ref[1, :, :1] + 1.0
    return lax.rsqrt(deg)


def _tc_mm1_body(x_ref, w_ref, h_ref):
    h_ref[...] = jnp.dot(x_ref[...], w_ref[...],
                         preferred_element_type=jnp.float32)


def _tc_scale_body(h_ref, degp_ref, hs_ref):
    hs_ref[...] = h_ref[...] * _dinv_from(degp_ref)


def _tc2_body(hs1_ref, aggp_ref, degp_ref, w2_ref, b1_ref, hs2_ref):
    dinv = _dinv_from(degp_ref)
    pre = (aggp_ref[0, 0] + aggp_ref[0, 1] + hs1_ref[...]) * dinv \
        + b1_ref[...]
    out1 = jnp.maximum(pre, 0.0)
    h2 = jnp.dot(out1, w2_ref[...], preferred_element_type=jnp.float32)
    hs2 = h2 * dinv
    hs2_ref[0] = hs2[:, :DA]
    hs2_ref[1] = hs2[:, DA:]


def _tc3_body(hs2_ref, aggp_ref, degp_ref, b2_ref, out_ref):
    dinv = _dinv_from(degp_ref)
    lo = (aggp_ref[0, 0] + aggp_ref[0, 1] + hs2_ref[0]) * dinv
    hi = (aggp_ref[1, 0] + aggp_ref[1, 1] + hs2_ref[1]) * dinv
    out_ref[...] = jnp.concatenate([lo, hi], axis=1) + b2_ref[...]


_deg_spec = pl.BlockSpec((NC, BN, DEG_W), lambda i: (0, i, 0))


def kernel(x, edge_index, W1, b1, W2, b2):
    ones16 = jnp.ones((CK, DEG_W), jnp.float32)
    zeros_deg = jnp.zeros((N_ACC, DEG_W), jnp.float32)
    zeros_da = jnp.zeros((N_ACC, DA), jnp.float32)

    degp = _deg_sc(edge_index, ones16, zeros_deg)

    h1 = pl.pallas_call(
        _tc_mm1_body,
        out_shape=jax.ShapeDtypeStruct((N, D_HID), jnp.float32),
    )(x, W1)

    hs1 = pl.pallas_call(
        _tc_scale_body,
        grid=(GRID,),
        in_specs=[
            pl.BlockSpec((BN, D_HID), lambda i: (i, 0)),
            _deg_spec,
        ],
        out_specs=pl.BlockSpec((BN, D_HID), lambda i: (i, 0)),
        out_shape=jax.ShapeDtypeStruct((N, D_HID), jnp.float32),
    )(h1, degp)

    agg1 = _agg_sc(hs1.reshape(1, N, DA), edge_index, zeros_da)

    hs2 = pl.pallas_call(
        _tc2_body,
        grid=(GRID,),
        in_specs=[
            pl.BlockSpec((BN, D_HID), lambda i: (i, 0)),
            pl.BlockSpec((1, NC, BN, DA), lambda i: (0, 0, i, 0)),
            _deg_spec,
            pl.BlockSpec((D_HID, D_OUT), lambda i: (0, 0)),
            pl.BlockSpec((1, D_HID), lambda i: (0, 0)),
        ],
        out_specs=pl.BlockSpec((2, BN, DA), lambda i: (0, i, 0)),
        out_shape=jax.ShapeDtypeStruct((2, N, DA), jnp.float32),
    )(hs1, agg1, degp, W2, b1.reshape(1, D_HID))

    agg2 = _agg_sc(hs2, edge_index, zeros_da)

    out = pl.pallas_call(
        _tc3_body,
        grid=(GRID,),
        in_specs=[
            pl.BlockSpec((2, BN, DA), lambda i: (0, i, 0)),
            pl.BlockSpec((2, NC, BN, DA), lambda i: (0, 0, i, 0)),
            _deg_spec,
            pl.BlockSpec((1, D_OUT), lambda i: (0, 0)),
        ],
        out_specs=pl.BlockSpec((BN, D_OUT), lambda i: (i, 0)),
        out_shape=jax.ShapeDtypeStruct((N, D_OUT), jnp.float32),
    )(hs2, agg2, degp, b2.reshape(1, D_OUT))

    return out
